# baseline (speedup 1.0000x reference)
import functools

import jax
import jax.numpy as jnp
from jax import lax
from jax.experimental import pallas as pl
from jax.experimental.pallas import tpu as pltpu
from jax.experimental.pallas import tpu_sc as plsc

TOPK = 1000
B, N, C = 32, 32768, 80
BN = 32768
NBLK = N // BN


def _maxarg_body(x_ref, ms_ref, mi_ref):
    x = x_ref[0]
    m = x[:, 0]
    ms_ref[0, 0] = m
    mi_ref[0, 0] = m + 1.0


def _max_argmax(x):
    ms, mi = pl.pallas_call(
        _maxarg_body,
        grid=(B * NBLK,),
        in_specs=[pl.BlockSpec((1, BN, C), lambda i: (i // NBLK, i % NBLK, 0))],
        out_specs=[
            pl.BlockSpec((1, 1, BN), lambda i: (i, 0, 0)),
            pl.BlockSpec((1, 1, BN), lambda i: (i, 0, 0)),
        ],
        out_shape=[
            jax.ShapeDtypeStruct((B * NBLK, 1, BN), jnp.float32),
            jax.ShapeDtypeStruct((B * NBLK, 1, BN), jnp.float32),
        ],
    )(x)
    return ms.reshape(B, N), mi.reshape(B, N)


NC_SC = 2
NS_SC = 16
L = 16
NV = N // L
NB1 = 2048
NB2 = 2048
NB3 = 1024
KPAD = 1024
KVREG = KPAD // L


def _find_bucket(hist_ref, nb, r, lane_iota):
    def body(t, carry):
        c = nb // L - 1 - t
        csum, found, bkt, cgt = carry
        tot = hist_ref[pl.ds(c * L, L)]
        for l in range(1, L):
            tot = tot + hist_ref[pl.ds(l * NB1 + c * L, L)]
        rt = jnp.flip(tot)
        cs = plsc.cumsum(rt) + csum
        crossed = cs >= r
        cm = plsc.cummax(crossed.astype(jnp.int32))
        has = jnp.max(cm) > 0
        j0 = 16 - jnp.sum(cm)
        new_bkt = c * L + 15 - j0
        new_cgt = csum + jnp.sum(jnp.where(cm == 0, rt, 0))
        is_new = jnp.logical_and(jnp.logical_not(found), has)
        bkt = jnp.where(is_new, new_bkt, bkt)
        cgt = jnp.where(is_new, new_cgt, cgt)
        found = jnp.logical_or(found, has)
        csum = csum + jnp.sum(tot)
        return csum, found, bkt, cgt

    init = (jnp.int32(0), jnp.bool_(False), jnp.int32(0), jnp.int32(0))
    _, _, bkt, cgt = lax.fori_loop(0, nb // L, body, init)
    return bkt, cgt


def _clear_hist(hist_ref, nwords):
    z = jnp.zeros((L,), jnp.int32)

    def body(i, _):
        for d in range(4):
            hist_ref[pl.ds((i * 4 + d) * L, L)] = z
        return 0

    lax.fori_loop(0, nwords // (4 * L), body, 0)


def _splat_u32(scalar_i32):
    return jnp.full((L,), scalar_i32, jnp.int32).astype(jnp.uint32)


def _sc_topk_body(ms_hbm, os_hbm, oi_hbm, kf, kb, hist, ck, ci, ct, osb, oib):
    wid = lax.axis_index("s") * NC_SC + lax.axis_index("c")
    lane_iota = lax.iota(jnp.int32, L)
    ones = jnp.ones((L,), jnp.int32)

    pltpu.sync_copy(ms_hbm.at[wid], kf)

    def conv(i, _):
        for d in range(4):
            o = (i * 4 + d) * L
            v = kf[pl.ds(o, L)]
            u = lax.bitcast_convert_type(v, jnp.uint32)
            key = jnp.where((u >> 31) == jnp.uint32(0),
                            u | jnp.uint32(0x80000000), ~u)
            kb[pl.ds(o, L)] = key
        return 0

    lax.fori_loop(0, NV // 4, conv, 0)

    _clear_hist(hist, L * NB1)

    def h1(i, _):
        for d in range(4):
            key = kb[pl.ds((i * 4 + d) * L, L)]
            digit = (key >> 21).astype(jnp.int32)
            plsc.addupdate_scatter(hist, [lane_iota * NB1 + digit], ones)
        return 0

    lax.fori_loop(0, NV // 4, h1, 0)
    b1, cgt1 = _find_bucket(hist, NB1, TOPK, lane_iota)
    r1 = TOPK - cgt1
    b1v = _splat_u32(b1)

    _clear_hist(hist, L * NB1)

    def h2(i, _):
        for d in range(4):
            key = kb[pl.ds((i * 4 + d) * L, L)]
            m = (key >> 21) == b1v
            digit = ((key >> 10) & jnp.uint32(0x7FF)).astype(jnp.int32)
            plsc.addupdate_scatter(hist, [lane_iota * NB1 + digit], ones, mask=m)
        return 0

    lax.fori_loop(0, NV // 4, h2, 0)
    b2, cgt2 = _find_bucket(hist, NB2, r1, lane_iota)
    r2 = r1 - cgt2
    p2 = (b1 << 11) | b2
    p2v = _splat_u32(p2)

    _clear_hist(hist, L * NB1)

    def h3(i, _):
        for d in range(4):
            key = kb[pl.ds((i * 4 + d) * L, L)]
            m = (key >> 10) == p2v
            digit = (key & jnp.uint32(0x3FF)).astype(jnp.int32)
            plsc.addupdate_scatter(hist, [lane_iota * NB1 + digit], ones, mask=m)
        return 0

    lax.fori_loop(0, NV // 4, h3, 0)
    b3, cgt3 = _find_bucket(hist, NB3, r2, lane_iota)
    r3 = r2 - cgt3

    t_i32 = (b1 << 21) | (b2 << 10) | b3
    tvec = _splat_u32(t_i32)
    base_eq = TOPK - r3

    def clr(i, _):
        ck[pl.ds(i * L, L)] = jnp.zeros((L,), jnp.uint32)
        ci[pl.ds(i * L, L)] = jnp.zeros((L,), jnp.int32)
        return 0

    lax.fori_loop(0, KVREG, clr, 0)

    def comp(i, carry):
        o_gt, n_eq = carry
        for d in range(2):
            o = i * 2 + d
            key = kb[pl.ds(o * L, L)]
            idxv = o * L + lane_iota
            mgt = key > tvec
            plsc.store_compressed(ck.at[pl.ds(o_gt, L)], key, mask=mgt)
            plsc.store_compressed(ci.at[pl.ds(o_gt, L)], idxv, mask=mgt)
            o_gt = o_gt + jnp.sum(mgt.astype(jnp.int32))
            meq = key == tvec
            ccnt = plsc.cumsum(meq.astype(jnp.int32))
            acc = jnp.logical_and(meq, (n_eq + ccnt) <= r3)
            plsc.store_compressed(ck.at[pl.ds(base_eq + n_eq, L)], key, mask=acc)
            plsc.store_compressed(ci.at[pl.ds(base_eq + n_eq, L)], idxv, mask=acc)
            n_eq = n_eq + jnp.sum(acc.astype(jnp.int32))
        return o_gt, n_eq

    lax.fori_loop(0, NV // 2, comp, (jnp.int32(0), jnp.int32(0)))

    def load_kv(v):
        return ck[pl.ds(v * L, L)], ci[pl.ds(v * L, L)]

    def store_kv(v, k, x):
        ck[pl.ds(v * L, L)] = k
        ci[pl.ds(v * L, L)] = x

    def sort_vreg(v, desc):
        k, x = load_kv(v)
        sk, sx = plsc.sort_key_val(k, x)
        dv = jnp.full((L,), desc)
        sk = jnp.where(dv, jnp.flip(sk), sk)
        sx = jnp.where(dv, jnp.flip(sx), sx)
        store_kv(v, sk, sx)

    def init_sort(v, _):
        sort_vreg(v, (v % 2) == 0)
        return 0

    lax.fori_loop(0, KVREG, init_sort, 0)

    for kk in (32, 64, 128, 256, 512, 1024):
        d = kk // 2
        while d >= L:
            j = d // L

            def stage(t, _, j=j, kk=kk):
                i = ((t & ~(j - 1)) << 1) | (t & (j - 1))
                p = i | j
                ak, av = load_kv(i)
                bk, bv = load_kv(p)
                rank_ab = jnp.logical_or(
                    ak > bk, jnp.logical_and(ak == bk, av < bv))
                descv = jnp.full((L,), ((i * L) & kk) == 0)
                sel = jnp.where(descv, rank_ab, jnp.logical_not(rank_ab))
                hk = jnp.where(sel, ak, bk)
                hv = jnp.where(sel, av, bv)
                lk = jnp.where(sel, bk, ak)
                lv = jnp.where(sel, bv, av)
                store_kv(i, hk, hv)
                store_kv(p, lk, lv)
                return 0

            lax.fori_loop(0, KVREG // 2, stage, 0)
            d //= 2

        def finish(v, _, kk=kk):
            sort_vreg(v, ((v * L) & kk) == 0)
            return 0

        lax.fori_loop(0, KVREG, finish, 0)

    def stash_keys(v, _):
        k = ck[pl.ds(v * L, L)]
        kf[pl.ds(v * L, L)] = lax.bitcast_convert_type(k, jnp.float32)
        return 0

    lax.fori_loop(0, KVREG, stash_keys, 0)

    def tie_pass(src, dst, parity):
        def body(v, _):
            base = v * L
            g = base + lane_iota
            if parity == 0:
                p = g ^ 1
            else:
                p = (g + 1) ^ 1
                p = jnp.maximum(p - 1, 0)
                p = jnp.minimum(p, KPAD - 1)
            k = ck[pl.ds(base, L)]
            kp = lax.bitcast_convert_type(
                plsc.load_gather(kf, [p]), jnp.uint32)
            x = src[pl.ds(base, L)]
            xp = plsc.load_gather(src, [p])
            meq = k == kp
            lower = g < p
            resolved = jnp.where(lower, jnp.minimum(x, xp), jnp.maximum(x, xp))
            dst[pl.ds(base, L)] = jnp.where(meq, resolved, x)
            return 0

        lax.fori_loop(0, KVREG, body, 0)

    for _ in range(4):
        tie_pass(ci, ct, 0)
        tie_pass(ct, ci, 1)

    def out(v, _):
        k, x = load_kv(v)
        pos = (k >> 31) != jnp.uint32(0)
        u = jnp.where(pos, k ^ jnp.uint32(0x80000000), ~k)
        osb[pl.ds(v * L, L)] = lax.bitcast_convert_type(u, jnp.float32)
        oib[pl.ds(v * L, L)] = x.astype(jnp.float32)
        return 0

    lax.fori_loop(0, KVREG, out, 0)
    pltpu.sync_copy(osb, os_hbm.at[wid])
    pltpu.sync_copy(oib, oi_hbm.at[wid])


def _sc_topk(ms):
    mesh = plsc.VectorSubcoreMesh(core_axis_name="c", subcore_axis_name="s")
    f = functools.partial(
        pl.kernel,
        mesh=mesh,
        compiler_params=pltpu.CompilerParams(needs_layout_passes=False),
        out_type=[
            jax.ShapeDtypeStruct((B, KPAD), jnp.float32),
            jax.ShapeDtypeStruct((B, KPAD), jnp.float32),
        ],
        scratch_types=[
            pltpu.VMEM((N,), jnp.float32),
            pltpu.VMEM((N,), jnp.uint32),
            pltpu.VMEM((L * NB1,), jnp.int32),
            pltpu.VMEM((KPAD,), jnp.uint32),
            pltpu.VMEM((KPAD,), jnp.int32),
            pltpu.VMEM((KPAD,), jnp.int32),
            pltpu.VMEM((KPAD,), jnp.float32),
            pltpu.VMEM((KPAD,), jnp.float32),
        ],
    )(_sc_topk_body)
    return f(ms)


@jax.jit
def kernel(cls_score_list):
    ms, mi = _max_argmax(cls_score_list)
    os_pad, oi_pad = _sc_topk(ms)
    return os_pad[:, :TOPK], oi_pad[:, :TOPK], mi


# speedup vs baseline: 1.9474x; 1.9474x over previous
import functools

import jax
import jax.numpy as jnp
from jax import lax
from jax.experimental import pallas as pl
from jax.experimental.pallas import tpu as pltpu
from jax.experimental.pallas import tpu_sc as plsc

TOPK = 1000
B, N, C = 32, 32768, 80
BN = 32768
NBLK = N // BN


def _maxarg_body(x_ref, ms_ref, mi_ref):
    x = x_ref[0]
    xt = x.T
    m = jnp.max(xt, axis=0)
    ii = lax.broadcasted_iota(jnp.int32, (C, BN), 0)
    idx = jnp.min(jnp.where(xt == m[None, :], ii, C), axis=0)
    ms_ref[0, 0] = m
    mi_ref[0, 0] = idx.astype(jnp.float32)


def _max_argmax(x):
    ms, mi = pl.pallas_call(
        _maxarg_body,
        grid=(B * NBLK,),
        in_specs=[pl.BlockSpec((1, BN, C), lambda i: (i // NBLK, i % NBLK, 0))],
        out_specs=[
            pl.BlockSpec((1, 1, BN), lambda i: (i, 0, 0)),
            pl.BlockSpec((1, 1, BN), lambda i: (i, 0, 0)),
        ],
        out_shape=[
            jax.ShapeDtypeStruct((B * NBLK, 1, BN), jnp.float32),
            jax.ShapeDtypeStruct((B * NBLK, 1, BN), jnp.float32),
        ],
    )(x)
    return ms.reshape(B, N), mi.reshape(B, N)


NC_SC = 2
NS_SC = 16
L = 16
NV = N // L
NB1 = 2048
NB2 = 2048
NB3 = 1024
KPAD = 1024
KVREG = KPAD // L


def _find_bucket(hist_ref, nb, r, lane_iota):
    def body(t, carry):
        c = nb // L - 1 - t
        csum, found, bkt, cgt = carry
        tot = hist_ref[pl.ds(c * L, L)]
        for l in range(1, L):
            tot = tot + hist_ref[pl.ds(l * NB1 + c * L, L)]
        rt = jnp.flip(tot)
        cs = plsc.cumsum(rt) + csum
        crossed = cs >= r
        cm = plsc.cummax(crossed.astype(jnp.int32))
        has = jnp.max(cm) > 0
        j0 = 16 - jnp.sum(cm)
        new_bkt = c * L + 15 - j0
        new_cgt = csum + jnp.sum(jnp.where(cm == 0, rt, 0))
        is_new = jnp.logical_and(jnp.logical_not(found), has)
        bkt = jnp.where(is_new, new_bkt, bkt)
        cgt = jnp.where(is_new, new_cgt, cgt)
        found = jnp.logical_or(found, has)
        csum = csum + jnp.sum(tot)
        return csum, found, bkt, cgt

    init = (jnp.int32(0), jnp.bool_(False), jnp.int32(0), jnp.int32(0))
    _, _, bkt, cgt = lax.fori_loop(0, nb // L, body, init)
    return bkt, cgt


def _clear_hist(hist_ref, nwords):
    z = jnp.zeros((L,), jnp.int32)

    def body(i, _):
        for d in range(4):
            hist_ref[pl.ds((i * 4 + d) * L, L)] = z
        return 0

    lax.fori_loop(0, nwords // (4 * L), body, 0)


def _splat_u32(scalar_i32):
    return jnp.full((L,), scalar_i32, jnp.int32).astype(jnp.uint32)


def _sc_topk_body(ms_hbm, os_hbm, oi_hbm, kf, kb, hist, ck, ci, ct, osb, oib):
    wid = lax.axis_index("s") * NC_SC + lax.axis_index("c")
    lane_iota = lax.iota(jnp.int32, L)
    ones = jnp.ones((L,), jnp.int32)

    pltpu.sync_copy(ms_hbm.at[wid], kf)

    def conv(i, _):
        for d in range(4):
            o = (i * 4 + d) * L
            v = kf[pl.ds(o, L)]
            u = lax.bitcast_convert_type(v, jnp.uint32)
            key = jnp.where((u >> 31) == jnp.uint32(0),
                            u | jnp.uint32(0x80000000), ~u)
            kb[pl.ds(o, L)] = key
        return 0

    lax.fori_loop(0, NV // 4, conv, 0)

    _clear_hist(hist, L * NB1)

    def h1(i, _):
        for d in range(4):
            key = kb[pl.ds((i * 4 + d) * L, L)]
            digit = (key >> 21).astype(jnp.int32)
            plsc.addupdate_scatter(hist, [lane_iota * NB1 + digit], ones)
        return 0

    lax.fori_loop(0, NV // 4, h1, 0)
    b1, cgt1 = _find_bucket(hist, NB1, TOPK, lane_iota)
    r1 = TOPK - cgt1
    b1v = _splat_u32(b1)

    _clear_hist(hist, L * NB1)

    def h2(i, _):
        for d in range(4):
            key = kb[pl.ds((i * 4 + d) * L, L)]
            m = (key >> 21) == b1v
            digit = ((key >> 10) & jnp.uint32(0x7FF)).astype(jnp.int32)
            plsc.addupdate_scatter(hist, [lane_iota * NB1 + digit], ones, mask=m)
        return 0

    lax.fori_loop(0, NV // 4, h2, 0)
    b2, cgt2 = _find_bucket(hist, NB2, r1, lane_iota)
    r2 = r1 - cgt2
    p2 = (b1 << 11) | b2
    p2v = _splat_u32(p2)

    _clear_hist(hist, L * NB1)

    def h3(i, _):
        for d in range(4):
            key = kb[pl.ds((i * 4 + d) * L, L)]
            m = (key >> 10) == p2v
            digit = (key & jnp.uint32(0x3FF)).astype(jnp.int32)
            plsc.addupdate_scatter(hist, [lane_iota * NB1 + digit], ones, mask=m)
        return 0

    lax.fori_loop(0, NV // 4, h3, 0)
    b3, cgt3 = _find_bucket(hist, NB3, r2, lane_iota)
    r3 = r2 - cgt3

    t_i32 = (b1 << 21) | (b2 << 10) | b3
    tvec = _splat_u32(t_i32)
    base_eq = TOPK - r3

    def clr(i, _):
        ck[pl.ds(i * L, L)] = jnp.zeros((L,), jnp.uint32)
        ci[pl.ds(i * L, L)] = jnp.zeros((L,), jnp.int32)
        return 0

    lax.fori_loop(0, KVREG, clr, 0)

    def comp(i, carry):
        o_gt, n_eq = carry
        for d in range(2):
            o = i * 2 + d
            key = kb[pl.ds(o * L, L)]
            idxv = o * L + lane_iota
            mgt = key > tvec
            plsc.store_compressed(ck.at[pl.ds(o_gt, L)], key, mask=mgt)
            plsc.store_compressed(ci.at[pl.ds(o_gt, L)], idxv, mask=mgt)
            o_gt = o_gt + jnp.sum(mgt.astype(jnp.int32))
            meq = key == tvec
            ccnt = plsc.cumsum(meq.astype(jnp.int32))
            acc = jnp.logical_and(meq, (n_eq + ccnt) <= r3)
            plsc.store_compressed(ck.at[pl.ds(base_eq + n_eq, L)], key, mask=acc)
            plsc.store_compressed(ci.at[pl.ds(base_eq + n_eq, L)], idxv, mask=acc)
            n_eq = n_eq + jnp.sum(acc.astype(jnp.int32))
        return o_gt, n_eq

    lax.fori_loop(0, NV // 2, comp, (jnp.int32(0), jnp.int32(0)))

    def load_kv(v):
        return ck[pl.ds(v * L, L)], ci[pl.ds(v * L, L)]

    def store_kv(v, k, x):
        ck[pl.ds(v * L, L)] = k
        ci[pl.ds(v * L, L)] = x

    def sort_vreg(v, desc):
        k, x = load_kv(v)
        sk, sx = plsc.sort_key_val(k, x)
        dv = jnp.full((L,), desc)
        sk = jnp.where(dv, jnp.flip(sk), sk)
        sx = jnp.where(dv, jnp.flip(sx), sx)
        store_kv(v, sk, sx)

    def init_sort(v, _):
        sort_vreg(v, (v % 2) == 0)
        return 0

    lax.fori_loop(0, KVREG, init_sort, 0)

    for kk in (32, 64, 128, 256, 512, 1024):
        d = kk // 2
        while d >= L:
            j = d // L

            def stage(t, _, j=j, kk=kk):
                i = ((t & ~(j - 1)) << 1) | (t & (j - 1))
                p = i | j
                ak, av = load_kv(i)
                bk, bv = load_kv(p)
                rank_ab = jnp.logical_or(
                    ak > bk, jnp.logical_and(ak == bk, av < bv))
                descv = jnp.full((L,), ((i * L) & kk) == 0)
                sel = jnp.where(descv, rank_ab, jnp.logical_not(rank_ab))
                hk = jnp.where(sel, ak, bk)
                hv = jnp.where(sel, av, bv)
                lk = jnp.where(sel, bk, ak)
                lv = jnp.where(sel, bv, av)
                store_kv(i, hk, hv)
                store_kv(p, lk, lv)
                return 0

            lax.fori_loop(0, KVREG // 2, stage, 0)
            d //= 2

        def finish(v, _, kk=kk):
            sort_vreg(v, ((v * L) & kk) == 0)
            return 0

        lax.fori_loop(0, KVREG, finish, 0)

    def stash_keys(v, _):
        k = ck[pl.ds(v * L, L)]
        kf[pl.ds(v * L, L)] = lax.bitcast_convert_type(k, jnp.float32)
        return 0

    lax.fori_loop(0, KVREG, stash_keys, 0)

    def tie_pass(src, dst, parity):
        def body(v, _):
            base = v * L
            g = base + lane_iota
            if parity == 0:
                p = g ^ 1
            else:
                p = (g + 1) ^ 1
                p = jnp.maximum(p - 1, 0)
                p = jnp.minimum(p, KPAD - 1)
            k = ck[pl.ds(base, L)]
            kp = lax.bitcast_convert_type(
                plsc.load_gather(kf, [p]), jnp.uint32)
            x = src[pl.ds(base, L)]
            xp = plsc.load_gather(src, [p])
            meq = k == kp
            lower = g < p
            resolved = jnp.where(lower, jnp.minimum(x, xp), jnp.maximum(x, xp))
            dst[pl.ds(base, L)] = jnp.where(meq, resolved, x)
            return 0

        lax.fori_loop(0, KVREG, body, 0)

    for _ in range(4):
        tie_pass(ci, ct, 0)
        tie_pass(ct, ci, 1)

    def out(v, _):
        k, x = load_kv(v)
        pos = (k >> 31) != jnp.uint32(0)
        u = jnp.where(pos, k ^ jnp.uint32(0x80000000), ~k)
        osb[pl.ds(v * L, L)] = lax.bitcast_convert_type(u, jnp.float32)
        oib[pl.ds(v * L, L)] = x.astype(jnp.float32)
        return 0

    lax.fori_loop(0, KVREG, out, 0)
    pltpu.sync_copy(osb, os_hbm.at[wid])
    pltpu.sync_copy(oib, oi_hbm.at[wid])


def _sc_topk(ms):
    mesh = plsc.VectorSubcoreMesh(core_axis_name="c", subcore_axis_name="s")
    f = functools.partial(
        pl.kernel,
        mesh=mesh,
        compiler_params=pltpu.CompilerParams(needs_layout_passes=False),
        out_type=[
            jax.ShapeDtypeStruct((B, KPAD), jnp.float32),
            jax.ShapeDtypeStruct((B, KPAD), jnp.float32),
        ],
        scratch_types=[
            pltpu.VMEM((N,), jnp.float32),
            pltpu.VMEM((N,), jnp.uint32),
            pltpu.VMEM((L * NB1,), jnp.int32),
            pltpu.VMEM((KPAD,), jnp.uint32),
            pltpu.VMEM((KPAD,), jnp.int32),
            pltpu.VMEM((KPAD,), jnp.int32),
            pltpu.VMEM((KPAD,), jnp.float32),
            pltpu.VMEM((KPAD,), jnp.float32),
        ],
    )(_sc_topk_body)
    return f(ms)


@jax.jit
def kernel(cls_score_list):
    ms, mi = _max_argmax(cls_score_list)
    os_pad, oi_pad = _sc_topk(ms)
    return os_pad[:, :TOPK], oi_pad[:, :TOPK], mi
